# baseline (speedup 1.0000x reference)
import functools

import jax
import jax.numpy as jnp
from jax import lax
from jax.experimental import pallas as pl
from jax.experimental.pallas import tpu as pltpu
from jax.experimental.pallas import tpu_sc as plsc

WORD_PAD = 100000
EMBED = 64
NUM_QW = 200
NC = 2
NS = 16
NW = NC * NS
LANES = 16
NCH = EMBED // LANES
G = 8
SPLITS = ((0, 128), (128, 72))


def _sc_to_bf16(word_table):
  V = word_table.shape[0]
  CR = 640
  nchunks = (V + CR - 1) // CR
  last_base = V - CR
  kmax = (nchunks + NW - 1) // NW
  assert nchunks >= 2 * NW

  mesh = plsc.VectorSubcoreMesh(core_axis_name="c", subcore_axis_name="s")

  @functools.partial(
      pl.kernel,
      out_type=jax.ShapeDtypeStruct((V, EMBED), jnp.bfloat16),
      mesh=mesh,
      compiler_params=pltpu.CompilerParams(
          use_tc_tiling_on_sc=False, needs_layout_passes=False),
      scratch_types=[
          (pltpu.VMEM((CR, EMBED), jnp.float32),) * 2,
          (pltpu.VMEM((CR, EMBED), jnp.bfloat16),) * 2,
          (pltpu.SemaphoreType.DMA,) * 2,
          (pltpu.SemaphoreType.DMA,) * 2,
      ],
  )
  def cvt_kernel(wt_hbm, out_hbm, in_v, out_v, in_sems, out_sems):
    wid = lax.axis_index("s") * NC + lax.axis_index("c")

    def chunk_base(k):
      return jnp.minimum((wid + NW * k) * CR, last_base)

    def fire_in(k):
      @pl.when(wid + NW * k < nchunks)
      def _():
        pltpu.make_async_copy(
            wt_hbm.at[pl.ds(chunk_base(k), CR)], in_v[k % 2],
            in_sems[k % 2]).start()

    fire_in(0)
    for k in range(kmax):
      cid = wid + NW * k

      @pl.when(cid < nchunks)
      def _(cid=cid, k=k):
        base = chunk_base(k)
        fire_in(k + 1)
        pltpu.make_async_copy(
            wt_hbm.at[pl.ds(base, CR)], in_v[k % 2], in_sems[k % 2]).wait()

        @pl.when(k >= 2)
        def _():
          pltpu.make_async_copy(
              out_v[k % 2], out_hbm.at[pl.ds(base, CR)],
              out_sems[k % 2]).wait()

        @pl.loop(0, CR, unroll=4)
        def _r(r):
          for h in range(2):
            a = in_v[k % 2][r, pl.ds(h * 2 * LANES, LANES)]
            b = in_v[k % 2][r, pl.ds(h * 2 * LANES + LANES, LANES)]
            out_v[k % 2][r, pl.ds(h * 2 * LANES, 2 * LANES)] = plsc.pack(
                a, b, format=plsc.PackFormat.INTERLEAVED)

        pltpu.make_async_copy(
            out_v[k % 2], out_hbm.at[pl.ds(base, CR)],
            out_sems[k % 2]).start()

    for p in range(2):
      pltpu.make_async_copy(
          out_v[p], out_hbm.at[pl.ds(0, CR)], out_sems[p]).wait()

  return cvt_kernel(word_table)


def _sc_pooled_sum(word_table, qw, B):
  rows_per_w = B // NW
  steps = rows_per_w // G
  assert steps % 2 == 0

  mesh = plsc.VectorSubcoreMesh(core_axis_name="c", subcore_axis_name="s")

  @functools.partial(
      pl.kernel,
      out_type=jax.ShapeDtypeStruct((B, EMBED), jnp.float32),
      mesh=mesh,
      compiler_params=pltpu.CompilerParams(
          use_tc_tiling_on_sc=False, needs_layout_passes=False),
      scratch_types=[
          pltpu.VMEM((2, G, NUM_QW), jnp.int32),
          pltpu.VMEM((G * NUM_QW, EMBED), jnp.bfloat16),
          pltpu.VMEM((G * NUM_QW, EMBED), jnp.bfloat16),
          pltpu.VMEM((2, G, EMBED), jnp.float32),
          (pltpu.SemaphoreType.DMA,) * 2,
          (pltpu.SemaphoreType.DMA,) * 2,
          (pltpu.SemaphoreType.DMA,) * 2,
      ],
  )
  def sc_kernel(wt_hbm, qw_hbm, out_hbm, idx_v, rows0, rows1, out_v,
                idx_sems, gat_sems, out_sems):
    wid = lax.axis_index("s") * NC + lax.axis_index("c")
    row0 = wid * rows_per_w
    rows_bufs = (rows0, rows1)

    def idx_start(step, buf):
      pltpu.async_copy(
          qw_hbm.at[pl.ds(row0 + step * G, G)], idx_v.at[buf],
          idx_sems[buf])

    def idx_wait(buf):
      pltpu.make_async_copy(
          qw_hbm.at[pl.ds(row0, G)], idx_v.at[buf], idx_sems[buf]).wait()

    def fire(buf):
      for g in range(G):
        for off, n in SPLITS:
          pltpu.async_copy(
              wt_hbm.at[idx_v.at[buf, g, pl.ds(off, n)]],
              rows_bufs[buf].at[pl.ds(g * NUM_QW + off, n)],
              gat_sems[buf])

    def drain(buf):
      for g in range(G):
        for off, n in SPLITS:
          pltpu.make_async_copy(
              wt_hbm.at[idx_v.at[buf, g, pl.ds(off, n)]],
              rows_bufs[buf].at[pl.ds(g * NUM_QW + off, n)],
              gat_sems[buf]).wait()

    def out_wait(buf):
      pltpu.make_async_copy(
          out_v.at[buf], out_hbm.at[pl.ds(row0, G)], out_sems[buf]).wait()

    idx_start(0, 0)
    idx_wait(0)
    fire(0)
    idx_start(1, 1)

    @pl.loop(0, steps, step=2)
    def _outer(s0):
      for b in range(2):
        s = s0 + b

        @pl.when(s + 1 < steps)
        def _(b=b):
          idx_wait(1 - b)
          fire(1 - b)

        drain(b)
        @pl.when(s + 2 < steps)
        def _(s=s, b=b):
          idx_start(s + 2, b)
        rows_ref = rows_bufs[b]
        zero = jnp.zeros((LANES,), jnp.float32)

        @pl.loop(0, NUM_QW, init_carry=(zero,) * (G * NCH), unroll=4)
        def accs(l, carry, rows_ref=rows_ref):
          carry = list(carry)
          for g in range(G):
            for h in range(2):
              x = rows_ref[g * NUM_QW + l, pl.ds(h * 2 * LANES, 2 * LANES)]
              ev, od = plsc.unpack(x, format=plsc.PackFormat.INTERLEAVED)
              carry[g * NCH + 2 * h] = carry[g * NCH + 2 * h] + ev
              carry[g * NCH + 2 * h + 1] = carry[g * NCH + 2 * h + 1] + od
          return tuple(carry)

        @pl.when(s >= 2)
        def _(b=b):
          out_wait(b)

        for g in range(G):
          for c in range(NCH):
            out_v[b, g, pl.ds(c * LANES, LANES)] = accs[g * NCH + c]
        pltpu.async_copy(
            out_v.at[b], out_hbm.at[pl.ds(row0 + s * G, G)], out_sems[b])

    out_wait(0)
    out_wait(1)

  return sc_kernel(word_table, qw)


def _tc_finish(pooled_sum, query_words, W_proj, b_proj):
  B = pooled_sum.shape[0]
  blk = 2048
  grid = B // blk

  def body(qw_ref, ps_ref, w_ref, b_ref, o_ref):
    vl = jnp.sum((qw_ref[...] != WORD_PAD).astype(jnp.float32), axis=1,
                 keepdims=True)
    pooled = ps_ref[...] / (vl + 1e-6)
    o_ref[...] = jnp.tanh(
        lax.dot_general(pooled, w_ref[...], (((1,), (1,)), ((), ())),
                        preferred_element_type=jnp.float32) + b_ref[...])

  return pl.pallas_call(
      body,
      out_shape=jax.ShapeDtypeStruct((B, EMBED), jnp.float32),
      grid=(grid,),
      in_specs=[
          pl.BlockSpec((blk, NUM_QW), lambda i: (i, 0)),
          pl.BlockSpec((blk, EMBED), lambda i: (i, 0)),
          pl.BlockSpec((EMBED, EMBED), lambda i: (0, 0)),
          pl.BlockSpec((1, EMBED), lambda i: (0, 0)),
      ],
      out_specs=pl.BlockSpec((blk, EMBED), lambda i: (i, 0)),
  )(query_words, pooled_sum, W_proj, b_proj.reshape(1, EMBED))


def kernel(items, query_words, word_table, W_proj, b_proj, item_table):
  del items, item_table
  B = query_words.shape[0]
  qw = query_words.astype(jnp.int32)
  wt16 = _sc_to_bf16(word_table)
  pooled_sum = _sc_pooled_sum(wt16, qw, B)
  return _tc_finish(pooled_sum, qw, W_proj, b_proj)


# speedup vs baseline: 1.3866x; 1.3866x over previous
import functools

import jax
import jax.numpy as jnp
from jax import lax
from jax.experimental import pallas as pl
from jax.experimental.pallas import tpu as pltpu
from jax.experimental.pallas import tpu_sc as plsc

WORD_PAD = 100000
EMBED = 64
NUM_QW = 200
NC = 2
NS = 16
NW = NC * NS
LANES = 16
NCH = EMBED // LANES
G = 8
SPLITS = ((0, 128), (128, 72))


def _sc_to_bf16(word_table):
  V = word_table.shape[0]
  CR = 640
  nchunks = (V + CR - 1) // CR
  last_base = V - CR
  kmax = (nchunks + NW - 1) // NW
  assert nchunks >= 2 * NW

  mesh = plsc.VectorSubcoreMesh(core_axis_name="c", subcore_axis_name="s")

  @functools.partial(
      pl.kernel,
      out_type=jax.ShapeDtypeStruct((V, EMBED), jnp.bfloat16),
      mesh=mesh,
      compiler_params=pltpu.CompilerParams(
          use_tc_tiling_on_sc=False, needs_layout_passes=False),
      scratch_types=[
          (pltpu.VMEM((CR, EMBED), jnp.float32),) * 2,
          (pltpu.VMEM((CR, EMBED), jnp.bfloat16),) * 2,
          (pltpu.SemaphoreType.DMA,) * 2,
          (pltpu.SemaphoreType.DMA,) * 2,
      ],
  )
  def cvt_kernel(wt_hbm, out_hbm, in_v, out_v, in_sems, out_sems):
    wid = lax.axis_index("s") * NC + lax.axis_index("c")

    def chunk_base(k):
      return jnp.minimum((wid + NW * k) * CR, last_base)

    def fire_in(k):
      @pl.when(wid + NW * k < nchunks)
      def _():
        pltpu.make_async_copy(
            wt_hbm.at[pl.ds(chunk_base(k), CR)], in_v[k % 2],
            in_sems[k % 2]).start()

    fire_in(0)
    for k in range(kmax):
      cid = wid + NW * k

      @pl.when(cid < nchunks)
      def _(cid=cid, k=k):
        base = chunk_base(k)
        fire_in(k + 1)
        pltpu.make_async_copy(
            wt_hbm.at[pl.ds(base, CR)], in_v[k % 2], in_sems[k % 2]).wait()

        @pl.when(k >= 2)
        def _():
          pltpu.make_async_copy(
              out_v[k % 2], out_hbm.at[pl.ds(base, CR)],
              out_sems[k % 2]).wait()

        @pl.loop(0, CR, unroll=4)
        def _r(r):
          for h in range(2):
            a = in_v[k % 2][r, pl.ds(h * 2 * LANES, LANES)]
            b = in_v[k % 2][r, pl.ds(h * 2 * LANES + LANES, LANES)]
            out_v[k % 2][r, pl.ds(h * 2 * LANES, 2 * LANES)] = plsc.pack(
                a, b, format=plsc.PackFormat.INTERLEAVED)

        pltpu.make_async_copy(
            out_v[k % 2], out_hbm.at[pl.ds(base, CR)],
            out_sems[k % 2]).start()

    for p in range(2):
      pltpu.make_async_copy(
          out_v[p], out_hbm.at[pl.ds(0, CR)], out_sems[p]).wait()

  return cvt_kernel(word_table)


def _sc_pooled_sum(word_table, qw, B):
  rows_per_w = B // NW
  steps = rows_per_w // G
  assert steps % 2 == 0

  mesh = plsc.VectorSubcoreMesh(core_axis_name="c", subcore_axis_name="s")

  @functools.partial(
      pl.kernel,
      out_type=jax.ShapeDtypeStruct((B, EMBED), jnp.float32),
      mesh=mesh,
      compiler_params=pltpu.CompilerParams(
          use_tc_tiling_on_sc=False, needs_layout_passes=False),
      scratch_types=[
          pltpu.VMEM((2, G, NUM_QW), jnp.int32),
          pltpu.VMEM((G * NUM_QW, EMBED), jnp.bfloat16),
          pltpu.VMEM((G * NUM_QW, EMBED), jnp.bfloat16),
          pltpu.VMEM((2, G, EMBED), jnp.float32),
          (pltpu.SemaphoreType.DMA,) * 2,
          (pltpu.SemaphoreType.DMA,) * 2,
          (pltpu.SemaphoreType.DMA,) * 2,
      ],
  )
  def sc_kernel(wt_hbm, qw_hbm, out_hbm, idx_v, rows0, rows1, out_v,
                idx_sems, gat_sems, out_sems):
    wid = lax.axis_index("s") * NC + lax.axis_index("c")
    row0 = wid * rows_per_w
    rows_bufs = (rows0, rows1)

    def idx_start(step, buf):
      pltpu.async_copy(
          qw_hbm.at[pl.ds(row0 + step * G, G)], idx_v.at[buf],
          idx_sems[buf])

    def idx_wait(buf):
      pltpu.make_async_copy(
          qw_hbm.at[pl.ds(row0, G)], idx_v.at[buf], idx_sems[buf]).wait()

    def fire(buf):
      for g in range(G):
        for off, n in SPLITS:
          pltpu.async_copy(
              wt_hbm.at[idx_v.at[buf, g, pl.ds(off, n)]],
              rows_bufs[buf].at[pl.ds(g * NUM_QW + off, n)],
              gat_sems[buf])

    def drain(buf):
      for g in range(G):
        for off, n in SPLITS:
          pltpu.make_async_copy(
              wt_hbm.at[idx_v.at[buf, g, pl.ds(off, n)]],
              rows_bufs[buf].at[pl.ds(g * NUM_QW + off, n)],
              gat_sems[buf]).wait()

    def out_wait(buf):
      pltpu.make_async_copy(
          out_v.at[buf], out_hbm.at[pl.ds(row0, G)], out_sems[buf]).wait()

    idx_start(0, 0)
    idx_wait(0)
    fire(0)
    idx_start(1, 1)

    @pl.loop(0, steps, step=2)
    def _outer(s0):
      for b in range(2):
        s = s0 + b

        @pl.when(s + 1 < steps)
        def _(b=b):
          idx_wait(1 - b)
          fire(1 - b)

        drain(b)
        @pl.when(s + 2 < steps)
        def _(s=s, b=b):
          idx_start(s + 2, b)
        rows_ref = rows_bufs[b]
        zero = jnp.zeros((LANES,), jnp.float32)

        @pl.loop(0, NUM_QW, init_carry=(zero,) * (G * NCH), unroll=2)
        def accs(l, carry, rows_ref=rows_ref):
          carry = list(carry)
          for g in range(G):
            for h in range(2):
              x = rows_ref[g * NUM_QW + l, pl.ds(h * 2 * LANES, 2 * LANES)]
              ev, od = plsc.unpack(x, format=plsc.PackFormat.INTERLEAVED)
              carry[g * NCH + 2 * h] = carry[g * NCH + 2 * h] + ev
              carry[g * NCH + 2 * h + 1] = carry[g * NCH + 2 * h + 1] + od
          return tuple(carry)

        @pl.when(s >= 2)
        def _(b=b):
          out_wait(b)

        for g in range(G):
          for c in range(NCH):
            out_v[b, g, pl.ds(c * LANES, LANES)] = accs[g * NCH + c]
        pltpu.async_copy(
            out_v.at[b], out_hbm.at[pl.ds(row0 + s * G, G)], out_sems[b])

    out_wait(0)
    out_wait(1)

  return sc_kernel(word_table, qw)


def _tc_finish(pooled_sum, query_words, W_proj, b_proj):
  B = pooled_sum.shape[0]
  blk = 2048
  grid = B // blk

  def body(qw_ref, ps_ref, w_ref, b_ref, o_ref):
    vl = jnp.sum((qw_ref[...] != WORD_PAD).astype(jnp.float32), axis=1,
                 keepdims=True)
    pooled = ps_ref[...] / (vl + 1e-6)
    o_ref[...] = jnp.tanh(
        lax.dot_general(pooled, w_ref[...], (((1,), (1,)), ((), ())),
                        preferred_element_type=jnp.float32) + b_ref[...])

  return pl.pallas_call(
      body,
      out_shape=jax.ShapeDtypeStruct((B, EMBED), jnp.float32),
      grid=(grid,),
      in_specs=[
          pl.BlockSpec((blk, NUM_QW), lambda i: (i, 0)),
          pl.BlockSpec((blk, EMBED), lambda i: (i, 0)),
          pl.BlockSpec((EMBED, EMBED), lambda i: (0, 0)),
          pl.BlockSpec((1, EMBED), lambda i: (0, 0)),
      ],
      out_specs=pl.BlockSpec((blk, EMBED), lambda i: (i, 0)),
  )(query_words, pooled_sum, W_proj, b_proj.reshape(1, EMBED))


def kernel(items, query_words, word_table, W_proj, b_proj, item_table):
  del items, item_table
  B = query_words.shape[0]
  qw = query_words.astype(jnp.int32)
  wt16 = _sc_to_bf16(word_table)
  pooled_sum = _sc_pooled_sum(wt16, qw, B)
  return _tc_finish(pooled_sum, qw, W_proj, b_proj)
